# baseline (speedup 1.0000x reference)
import functools

import jax
import jax.numpy as jnp
from jax import lax
from jax.experimental import pallas as pl
from jax.experimental.pallas import tpu as pltpu
from jax.experimental.pallas import tpu_sc as plsc

B, F, N, K = 8, 64, 256, 512
NC, NS = 2, 16
NW = NC * NS
TOK = B * N
TPW = TOK // NW


def _dist_body(x_ref, emb_ref, dist_ref, idx_ref):
    xb = x_ref[0]
    emb = emb_ref[...]
    xtb = xb.T
    embt = emb.T
    dot = lax.dot_general(xtb, embt, (((1,), (0,)), ((), ())),
                          precision=lax.Precision.HIGHEST,
                          preferred_element_type=jnp.float32)
    c2 = jnp.sum(embt * embt, axis=0, keepdims=True)
    x2 = jnp.sum(xtb * xtb, axis=1, keepdims=True)
    g = c2 - 2.0 * dot
    dist_ref[0] = g + x2
    minv = jnp.min(g, axis=1, keepdims=True)
    iota = lax.broadcasted_iota(jnp.int32, (N, K), 1)
    idx_ref[0] = jnp.min(jnp.where(g == minv, iota, K), axis=1, keepdims=True)


def _transpose_body(qnf_ref, q_ref):
    q_ref[0] = qnf_ref[0].T


_sc_mesh = plsc.VectorSubcoreMesh(core_axis_name="c", subcore_axis_name="s")


@functools.partial(
    pl.kernel,
    mesh=_sc_mesh,
    out_type=jax.ShapeDtypeStruct((TOK, F), jnp.float32),
    scratch_types=[
        pltpu.VMEM((TPW,), jnp.int32),
        pltpu.VMEM((TPW, F), jnp.float32),
        pltpu.SemaphoreType.DMA,
    ],
    compiler_params=pltpu.CompilerParams(use_tc_tiling_on_sc=False),
)
def _sc_gather(table_hbm, idx_hbm, out_hbm, idx_v, rows_v, sem):
    wid = lax.axis_index("s") * NC + lax.axis_index("c")
    base = wid * TPW
    pltpu.sync_copy(idx_hbm.at[pl.ds(base, TPW)], idx_v)
    pltpu.async_copy(table_hbm.at[idx_v], rows_v, sem).wait()
    pltpu.sync_copy(rows_v, out_hbm.at[pl.ds(base, TPW)])


def kernel(x, emb_weight):
    dist, idx = pl.pallas_call(
        _dist_body,
        grid=(B,),
        in_specs=[
            pl.BlockSpec((1, F, N), lambda b: (b, 0, 0)),
            pl.BlockSpec((K, F), lambda b: (0, 0)),
        ],
        out_specs=[
            pl.BlockSpec((1, N, K), lambda b: (b, 0, 0)),
            pl.BlockSpec((1, N, 1), lambda b: (b, 0, 0)),
        ],
        out_shape=[
            jax.ShapeDtypeStruct((B, N, K), jnp.float32),
            jax.ShapeDtypeStruct((B, N, 1), jnp.int32),
        ],
    )(x, emb_weight)
    q_nf = _sc_gather(emb_weight, idx.reshape(TOK))
    q = pl.pallas_call(
        _transpose_body,
        grid=(B,),
        in_specs=[pl.BlockSpec((1, N, F), lambda b: (b, 0, 0))],
        out_specs=pl.BlockSpec((1, F, N), lambda b: (b, 0, 0)),
        out_shape=jax.ShapeDtypeStruct((B, F, N), jnp.float32),
    )(q_nf.reshape(B, N, F))
    return q, dist


# speedup vs baseline: 2.5173x; 2.5173x over previous
import jax
import jax.numpy as jnp
from jax import lax
from jax.experimental import pallas as pl

B, F, N, K = 8, 64, 256, 512


def _vq_body(x_ref, emb_ref, q_ref, dist_ref):
    xb = x_ref[0]
    emb = emb_ref[...]
    dotT = lax.dot_general(emb, xb, (((1,), (0,)), ((), ())),
                           precision=lax.Precision.HIGHEST,
                           preferred_element_type=jnp.float32)
    c2 = jnp.sum(emb * emb, axis=1, keepdims=True)
    x2 = jnp.sum(xb * xb, axis=0, keepdims=True)
    gT = c2 - 2.0 * dotT
    dist_ref[0] = (gT + x2).T
    minv = jnp.min(gT, axis=0, keepdims=True)
    iota = lax.broadcasted_iota(jnp.int32, (K, N), 0)
    idx = jnp.min(jnp.where(gT == minv, iota, K), axis=0, keepdims=True)
    ohT = (iota == idx).astype(jnp.float32)
    q_ref[0] = lax.dot_general(emb, ohT, (((0,), (0,)), ((), ())),
                               precision=lax.Precision.HIGHEST,
                               preferred_element_type=jnp.float32)


def kernel(x, emb_weight):
    q, dist = pl.pallas_call(
        _vq_body,
        grid=(B,),
        in_specs=[
            pl.BlockSpec((1, F, N), lambda b: (b, 0, 0)),
            pl.BlockSpec((K, F), lambda b: (0, 0)),
        ],
        out_specs=[
            pl.BlockSpec((1, F, N), lambda b: (b, 0, 0)),
            pl.BlockSpec((1, N, K), lambda b: (b, 0, 0)),
        ],
        out_shape=[
            jax.ShapeDtypeStruct((B, F, N), jnp.float32),
            jax.ShapeDtypeStruct((B, N, K), jnp.float32),
        ],
    )(x, emb_weight)
    return q, dist


# speedup vs baseline: 3.4987x; 1.3899x over previous
import jax
import jax.numpy as jnp
from jax import lax
from jax.experimental import pallas as pl

B, F, N, K = 8, 64, 256, 512


def _split(a):
    hi = a.astype(jnp.bfloat16)
    lo = (a - hi.astype(jnp.float32)).astype(jnp.bfloat16)
    return hi, lo


def _bdot(a, b, dims):
    return lax.dot_general(a, b, (dims, ((), ())),
                           preferred_element_type=jnp.float32)


def _vq_body(x_ref, emb_ref, q_ref, dist_ref):
    xb = x_ref[0]
    emb = emb_ref[...]
    xh, xl = _split(xb)
    eh, el = _split(emb)
    cd = ((1,), (0,))
    dotT = (_bdot(eh, xh, cd) + _bdot(eh, xl, cd)
            + _bdot(el, xh, cd))
    c2 = jnp.sum(emb * emb, axis=1, keepdims=True)
    x2 = jnp.sum(xb * xb, axis=0, keepdims=True)
    gT = c2 - 2.0 * dotT
    dist_ref[0] = (gT + x2).T
    minv = jnp.min(gT, axis=0, keepdims=True)
    iota = lax.broadcasted_iota(jnp.int32, (K, N), 0)
    idx = jnp.min(jnp.where(gT == minv, iota, K), axis=0, keepdims=True)
    ohT = (iota == idx).astype(jnp.bfloat16)
    cq = ((0,), (0,))
    q_ref[0] = _bdot(eh, ohT, cq) + _bdot(el, ohT, cq)


def kernel(x, emb_weight):
    q, dist = pl.pallas_call(
        _vq_body,
        grid=(B,),
        in_specs=[
            pl.BlockSpec((1, F, N), lambda b: (b, 0, 0)),
            pl.BlockSpec((K, F), lambda b: (0, 0)),
        ],
        out_specs=[
            pl.BlockSpec((1, F, N), lambda b: (b, 0, 0)),
            pl.BlockSpec((1, N, K), lambda b: (b, 0, 0)),
        ],
        out_shape=[
            jax.ShapeDtypeStruct((B, F, N), jnp.float32),
            jax.ShapeDtypeStruct((B, N, K), jnp.float32),
        ],
    )(x, emb_weight)
    return q, dist
